# baseline (speedup 1.0000x reference)
import functools

import jax
import jax.numpy as jnp
from jax import lax
from jax.experimental import pallas as pl
from jax.experimental.pallas import tpu as pltpu
from jax.experimental.pallas import tpu_sc as plsc

B = 16384
D = 64
F = 16
H1 = 128
H2 = 64

NC = 2
NS = 16
NW = NC * NS
BPW = B // NW
CHUNK = 128
NCHUNK = BPW // CHUNK

_sc_mesh = plsc.VectorSubcoreMesh(core_axis_name="c", subcore_axis_name="s")


@functools.partial(
    pl.kernel,
    mesh=_sc_mesh,
    compiler_params=pltpu.CompilerParams(use_tc_tiling_on_sc=False),
    out_type=[
        jax.ShapeDtypeStruct((B, D), jnp.float32),
        jax.ShapeDtypeStruct((B, D), jnp.float32),
    ],
    scratch_types=[
        pltpu.VMEM((BPW,), jnp.int32),
        pltpu.VMEM((BPW,), jnp.int32),
        pltpu.VMEM((BPW, D), jnp.float32),
        pltpu.VMEM((BPW, D), jnp.float32),
        pltpu.SemaphoreType.DMA,
        pltpu.SemaphoreType.DMA,
    ],
)
def _gather_sc(uids_hbm, mids_hbm, utab_hbm, mtab_hbm, uout_hbm, mout_hbm,
               uidx_v, midx_v, urows_v, mrows_v, usem, msem):
    wid = lax.axis_index("s") * NC + lax.axis_index("c")
    base = wid * BPW
    pltpu.sync_copy(uids_hbm.at[pl.ds(base, BPW)], uidx_v)
    pltpu.sync_copy(mids_hbm.at[pl.ds(base, BPW)], midx_v)
    ucopies = []
    mcopies = []
    for j in range(NCHUNK):
        sl = pl.ds(j * CHUNK, CHUNK)
        ucopies.append(
            pltpu.async_copy(utab_hbm.at[uidx_v.at[sl]], urows_v.at[sl], usem))
        mcopies.append(
            pltpu.async_copy(mtab_hbm.at[midx_v.at[sl]], mrows_v.at[sl], msem))
    for c in ucopies:
        c.wait()
    pltpu.sync_copy(urows_v, uout_hbm.at[pl.ds(base, BPW)])
    for c in mcopies:
        c.wait()
    pltpu.sync_copy(mrows_v, mout_hbm.at[pl.ds(base, BPW)])


BB = 512


def _mlp_body(u_ref, m_ref, f_ref, w1_ref, b1_ref, w2_ref, b2_ref,
              w3t_ref, b3_ref, o_ref):
    w1 = w1_ref[...]
    h1 = (
        jnp.dot(u_ref[...], w1[0:D, :], preferred_element_type=jnp.float32)
        + jnp.dot(m_ref[...], w1[D:2 * D, :], preferred_element_type=jnp.float32)
        + jnp.dot(f_ref[...], w1[2 * D:2 * D + F, :],
                  preferred_element_type=jnp.float32)
        + b1_ref[...]
    )
    h1 = jnp.maximum(h1, 0.0)
    h2 = jnp.maximum(
        jnp.dot(h1, w2_ref[...], preferred_element_type=jnp.float32)
        + b2_ref[...], 0.0)
    z = jnp.sum(h2 * w3t_ref[...], axis=1, keepdims=True) + b3_ref[...]
    o_ref[...] = jax.nn.sigmoid(z)


@jax.jit
def _mlp_tc(u, m, f, W1, b1, W2, b2, W3, b3):
    grid = (B // BB,)
    return pl.pallas_call(
        _mlp_body,
        grid=grid,
        in_specs=[
            pl.BlockSpec((BB, D), lambda i: (i, 0)),
            pl.BlockSpec((BB, D), lambda i: (i, 0)),
            pl.BlockSpec((BB, F), lambda i: (i, 0)),
            pl.BlockSpec((2 * D + F, H1), lambda i: (0, 0)),
            pl.BlockSpec((1, H1), lambda i: (0, 0)),
            pl.BlockSpec((H1, H2), lambda i: (0, 0)),
            pl.BlockSpec((1, H2), lambda i: (0, 0)),
            pl.BlockSpec((1, H2), lambda i: (0, 0)),
            pl.BlockSpec((1, 1), lambda i: (0, 0)),
        ],
        out_specs=pl.BlockSpec((BB, 1), lambda i: (i, 0)),
        out_shape=jax.ShapeDtypeStruct((B, 1), jnp.float32),
    )(u, m, f, W1, b1.reshape(1, H1), W2, b2.reshape(1, H2),
      W3.reshape(1, H2), b3.reshape(1, 1))


def kernel(user_ids, movie_ids, movie_features, user_table, movie_table,
           W1, b1, W2, b2, W3, b3):
    ue, me = _gather_sc(user_ids, movie_ids, user_table, movie_table)
    return _mlp_tc(ue, me, movie_features, W1, b1, W2, b2, W3, b3)


# speedup vs baseline: 1.6147x; 1.6147x over previous
import functools

import jax
import jax.numpy as jnp
from jax import lax
from jax.experimental import pallas as pl
from jax.experimental.pallas import tpu as pltpu
from jax.experimental.pallas import tpu_sc as plsc

B = 16384
D = 64
F = 16
H1 = 128
H2 = 64

NC = 2
NS = 16
NW = NC * NS
BPW = B // NW

_sc_mesh = plsc.VectorSubcoreMesh(core_axis_name="c", subcore_axis_name="s")


def _gather_one(ids_hbm, tab_hbm, out_hbm, base, idx_v, rows_v, sem):
    pltpu.sync_copy(ids_hbm.at[pl.ds(base, BPW)], idx_v)

    def _grp(g, carry):
        vec = idx_v[pl.ds(g * 16, 16)]
        for j in range(16):
            rid = vec[j]
            pltpu.async_copy(tab_hbm.at[pl.ds(rid, 1)],
                             rows_v.at[pl.ds(g * 16 + j, 1)], sem)
        return carry

    lax.fori_loop(0, BPW // 16, _grp, 0)
    pltpu.make_async_copy(out_hbm.at[pl.ds(base, BPW)], rows_v, sem).wait()
    pltpu.sync_copy(rows_v, out_hbm.at[pl.ds(base, BPW)])


@functools.partial(
    pl.kernel,
    mesh=_sc_mesh,
    out_type=[
        jax.ShapeDtypeStruct((B, D), jnp.float32),
        jax.ShapeDtypeStruct((B, D), jnp.float32),
    ],
    scratch_types=[
        pltpu.VMEM((BPW,), jnp.int32),
        pltpu.VMEM((BPW, D), jnp.float32),
        pltpu.SemaphoreType.DMA,
    ],
)
def _gather_sc(uids_hbm, mids_hbm, utab_hbm, mtab_hbm, uout_hbm, mout_hbm,
               idx_v, rows_v, sem):
    wid = lax.axis_index("s") * NC + lax.axis_index("c")
    base = wid * BPW
    _gather_one(uids_hbm, utab_hbm, uout_hbm, base, idx_v, rows_v, sem)
    _gather_one(mids_hbm, mtab_hbm, mout_hbm, base, idx_v, rows_v, sem)


BB = 512


def _mlp_body(u_ref, m_ref, f_ref, w1_ref, b1_ref, w2_ref, b2_ref,
              w3t_ref, b3_ref, o_ref):
    w1 = w1_ref[...]
    h1 = (
        jnp.dot(u_ref[...], w1[0:D, :], preferred_element_type=jnp.float32)
        + jnp.dot(m_ref[...], w1[D:2 * D, :], preferred_element_type=jnp.float32)
        + jnp.dot(f_ref[...], w1[2 * D:2 * D + F, :],
                  preferred_element_type=jnp.float32)
        + b1_ref[...]
    )
    h1 = jnp.maximum(h1, 0.0)
    h2 = jnp.maximum(
        jnp.dot(h1, w2_ref[...], preferred_element_type=jnp.float32)
        + b2_ref[...], 0.0)
    z = jnp.sum(h2 * w3t_ref[...], axis=1, keepdims=True) + b3_ref[...]
    o_ref[...] = jax.nn.sigmoid(z)


@jax.jit
def _mlp_tc(u, m, f, W1, b1, W2, b2, W3, b3):
    grid = (B // BB,)
    return pl.pallas_call(
        _mlp_body,
        grid=grid,
        in_specs=[
            pl.BlockSpec((BB, D), lambda i: (i, 0)),
            pl.BlockSpec((BB, D), lambda i: (i, 0)),
            pl.BlockSpec((BB, F), lambda i: (i, 0)),
            pl.BlockSpec((2 * D + F, H1), lambda i: (0, 0)),
            pl.BlockSpec((1, H1), lambda i: (0, 0)),
            pl.BlockSpec((H1, H2), lambda i: (0, 0)),
            pl.BlockSpec((1, H2), lambda i: (0, 0)),
            pl.BlockSpec((1, H2), lambda i: (0, 0)),
            pl.BlockSpec((1, 1), lambda i: (0, 0)),
        ],
        out_specs=pl.BlockSpec((BB, 1), lambda i: (i, 0)),
        out_shape=jax.ShapeDtypeStruct((B, 1), jnp.float32),
    )(u, m, f, W1, b1.reshape(1, H1), W2, b2.reshape(1, H2),
      W3.reshape(1, H2), b3.reshape(1, 1))


def kernel(user_ids, movie_ids, movie_features, user_table, movie_table,
           W1, b1, W2, b2, W3, b3):
    ue, me = _gather_sc(user_ids, movie_ids, user_table, movie_table)
    return _mlp_tc(ue, me, movie_features, W1, b1, W2, b2, W3, b3)
